# baseline (speedup 1.0000x reference)
import functools

import jax
import jax.numpy as jnp
from jax import lax
from jax.experimental import pallas as pl
from jax.experimental.pallas import tpu as pltpu
from jax.experimental.pallas import tpu_sc as plsc

N = 50000
D = 64
QH = 16
NQ = D // QH
NC, NS = 2, 16

K = 128
CPS = 16
SUP = K * CPS
NSUP = 26
E_T = SUP * NSUP
E_PAD = E_T * NS

AGG_ROWS = N + 8
NR = N // NC
HIST_ROWS = NR + 8

BR = 5000


def _zero_rows(ref, n_cols16):
    def body(r, _):
        for j in range(n_cols16):
            ref[r, pl.ds(16 * j, 16)] = jnp.zeros((16,), jnp.float32)
        return 0
    lax.fori_loop(0, K, body, 0)


@functools.partial(
    pl.kernel,
    out_type=jax.ShapeDtypeStruct((N, D), jnp.float32),
    mesh=plsc.VectorSubcoreMesh(core_axis_name="c", subcore_axis_name="s"),
    scratch_types=[
        pltpu.VMEM((2, CPS, K), jnp.int32),
        pltpu.VMEM((2, CPS, K), jnp.int32),
        pltpu.VMEM((2, CPS, K, QH), jnp.float32),
        pltpu.VMEM((K, QH), jnp.float32),
        pltpu.VMEM_SHARED((AGG_ROWS, QH), jnp.float32),
        pltpu.SemaphoreType.DMA,
        pltpu.SemaphoreType.DMA,
        pltpu.SemaphoreType.DMA,
        pltpu.SemaphoreType.DMA,
    ],
    compiler_params=pltpu.CompilerParams(use_tc_tiling_on_sc=False),
)
def _segsum_sc(q, srcs, dstb, rout, sv, dv, rows, zrow, agg, g0, g1, s0, s1):
    c = lax.axis_index("c")
    s = lax.axis_index("s")
    gsem = (g0, g1)
    ssem = (s0, s1)

    _zero_rows(zrow, 1)
    base_z = jnp.minimum(s * 3200, AGG_ROWS - 25 * K)
    base_w = jnp.minimum(s * 3128, N - 3128)

    for qi in range(2):
        qt = c * 2 + qi
        qcols = pl.ds(qt * QH, QH)

        def fire_gather(sup, b):
            pltpu.sync_copy(srcs.at[qt, s, sup], sv.at[b])
            pltpu.sync_copy(dstb.at[s, sup], dv.at[b])
            for j in range(CPS):
                pltpu.async_copy(q.at[sv.at[b, j]], rows.at[b, j], gsem[b])

        def drain_gather(b):
            for j in range(CPS):
                pltpu.make_async_copy(q.at[sv.at[b, j]], rows.at[b, j],
                                      gsem[b]).wait()

        def fire_scatter(b):
            for j in range(CPS):
                pltpu.async_copy(rows.at[b, j], agg.at[dv.at[b, j]], ssem[b],
                                 add=True)

        def drain_scatter(b):
            for j in range(CPS):
                pltpu.make_async_copy(rows.at[b, j], agg.at[dv.at[b, j]],
                                      ssem[b]).wait()

        def zero_chunk(k, _):
            pltpu.sync_copy(zrow, agg.at[pl.ds(base_z + k * K, K)])
            return 0
        lax.fori_loop(0, 25, zero_chunk, 0)
        plsc.subcore_barrier()

        def super_chunk(sup, _):
            pltpu.sync_copy(srcs.at[qt, s, sup], sv.at[0])
            pltpu.sync_copy(dstb.at[s, sup], dv.at[0])
            copies = []
            for j in range(CPS):
                copies.append(pltpu.async_copy(q.at[sv.at[0, j]],
                                               rows.at[0, j], gsem[0]))
            for cp in copies:
                cp.wait()
            for j in range(CPS):
                pltpu.sync_copy(rows.at[0, j], agg.at[dv.at[0, j]], add=True)
            return 0
        lax.fori_loop(0, NSUP, super_chunk, 0)
        plsc.subcore_barrier()

        pltpu.sync_copy(agg.at[pl.ds(base_w, 3128)],
                        rout.at[pl.ds(base_w, 3128), qcols])
        plsc.subcore_barrier()


@functools.partial(
    pl.kernel,
    out_type=jax.ShapeDtypeStruct((N, 16), jnp.float32),
    mesh=plsc.VectorSubcoreMesh(core_axis_name="c", subcore_axis_name="s"),
    scratch_types=[
        pltpu.VMEM((2, CPS, K), jnp.int32),
        pltpu.VMEM((SUP, 16), jnp.float32),
        pltpu.VMEM((K, 16), jnp.float32),
        pltpu.VMEM_SHARED((HIST_ROWS, 16), jnp.float32),
        pltpu.SemaphoreType.DMA,
        pltpu.SemaphoreType.DMA,
    ],
    compiler_params=pltpu.CompilerParams(use_tc_tiling_on_sc=False),
)
def _deg_sc(dstb, dout, dv, onesb, zrow, hist, s0, s1):
    c = lax.axis_index("c")
    s = lax.axis_index("s")
    ssem = (s0, s1)

    def fill_ones(r, _):
        onesb[r, pl.ds(0, 16)] = jnp.ones((16,), jnp.float32)
        return 0
    lax.fori_loop(0, SUP, fill_ones, 0)
    _zero_rows(zrow, 1)

    base_z = jnp.minimum(s * 1568, HIST_ROWS - 13 * K)

    def zero_chunk(k, _):
        pltpu.sync_copy(zrow, hist.at[pl.ds(base_z + k * K, K)])
        return 0
    lax.fori_loop(0, 13, zero_chunk, 0)
    plsc.subcore_barrier()

    lo = c * NR

    def load_remap(sup, b):
        pltpu.sync_copy(dstb.at[s, sup], dv.at[b])

        def remap(t, _):
            def remap16(u, _):
                v = dv[b, t, pl.ds(u * 16, 16)] - lo
                ok = (v >= 0) & (v < NR)
                dv[b, t, pl.ds(u * 16, 16)] = jnp.where(ok, v, NR)
                return 0
            lax.fori_loop(0, K // 16, remap16, 0)
            return 0
        lax.fori_loop(0, CPS, remap, 0)

    def fire_scatter(b):
        for j in range(CPS):
            pltpu.async_copy(onesb.at[pl.ds(j * K, K)], hist.at[dv.at[b, j]],
                             ssem[b], add=True)

    def drain_scatter(b):
        for j in range(CPS):
            pltpu.make_async_copy(onesb.at[pl.ds(j * K, K)],
                                  hist.at[dv.at[b, j]], ssem[b]).wait()

    load_remap(0, 0)

    def pair(g2, _):
        for b in (0, 1):
            sup = 2 * g2 + b
            if b == 0:
                fire_scatter(0)
                load_remap(sup + 1, 1)
                drain_scatter(0)
            else:
                fire_scatter(1)

                @pl.when(sup + 1 < NSUP)
                def _():
                    load_remap(sup + 1, 0)
                drain_scatter(1)
        return 0
    lax.fori_loop(0, NSUP // 2, pair, 0)
    plsc.subcore_barrier()

    base_w = jnp.minimum(s * 1568, NR - 1568)
    pltpu.sync_copy(hist.at[pl.ds(base_w, 1568)],
                    dout.at[pl.ds(c * NR + base_w, 1568)])


def _store_quarters(q_ref, q):
    for k in range(NQ):
        q_ref[k] = q[:, QH * k:QH * (k + 1)]


def _mm_first_body(x_ref, ws_ref, wn_ref, b_ref, p_ref, q_ref):
    h = x_ref[...]
    p_ref[...] = (jnp.dot(h, ws_ref[...], preferred_element_type=jnp.float32)
                  + b_ref[...])
    _store_quarters(q_ref, jnp.dot(h, wn_ref[...],
                                   preferred_element_type=jnp.float32))


def _mean_from(r_ref, d_ref):
    deg = jnp.maximum(d_ref[:, 0:1], 1.0)
    return r_ref[...] / deg


def _fuse_body(p_ref, r_ref, d_ref, ws_ref, wn_ref, b_ref, f_ref, po_ref, qo_ref):
    z = p_ref[...] + _mean_from(r_ref, d_ref)
    h = jnp.where(f_ref[0, 0] > 0.0, jnp.maximum(z, 0.0), z)
    po_ref[...] = (jnp.dot(h, ws_ref[...], preferred_element_type=jnp.float32)
                   + b_ref[...])
    _store_quarters(qo_ref, jnp.dot(h, wn_ref[...],
                                    preferred_element_type=jnp.float32))


_P_SPEC = pl.BlockSpec((BR, D), lambda i: (i, 0))
_Q_SPEC = pl.BlockSpec((NQ, BR, QH), lambda i: (0, i, 0))
_DEG_SPEC = pl.BlockSpec((BR, 16), lambda i: (i, 0))
_W_SPEC = pl.BlockSpec((D, D), lambda i: (0, 0))
_B_SPEC = pl.BlockSpec((1, D), lambda i: (0, 0))
_F_SPEC = pl.BlockSpec((1, 1), lambda i: (0, 0))
_PQ_SHAPES = [jax.ShapeDtypeStruct((N, D), jnp.float32),
              jax.ShapeDtypeStruct((NQ, N, QH), jnp.float32)]


def _tc_first(x, ws, wn, b1):
    return pl.pallas_call(
        _mm_first_body,
        grid=(N // BR,),
        in_specs=[_P_SPEC, _W_SPEC, _W_SPEC, _B_SPEC],
        out_specs=[_P_SPEC, _Q_SPEC],
        out_shape=_PQ_SHAPES,
    )(x, ws, wn, b1)


def _tc_fuse(p, r, deg, ws, wn, b1, flag):
    return pl.pallas_call(
        _fuse_body,
        grid=(N // BR,),
        in_specs=[_P_SPEC, _P_SPEC, _DEG_SPEC, _W_SPEC, _W_SPEC, _B_SPEC,
                  _F_SPEC],
        out_specs=[_P_SPEC, _Q_SPEC],
        out_shape=_PQ_SHAPES,
    )(p, r, deg, ws, wn, b1, flag)


def kernel(x, edge_index, Ws, Wn, b):
    src = edge_index[0].astype(jnp.int32)
    dst = edge_index[1].astype(jnp.int32)
    pad = E_PAD - src.shape[0]
    src_p = jnp.concatenate([src, jnp.zeros((pad,), jnp.int32)])
    dst_p = jnp.concatenate([dst, jnp.full((pad,), N, jnp.int32)])
    srcq = (src_p[None, :]
            + (jnp.arange(NQ, dtype=jnp.int32) * N)[:, None]
            ).reshape(NQ, NS, NSUP, CPS, K)
    dst_b = dst_p.reshape(NS, NSUP, CPS, K)

    deg = _deg_sc(dst_b)
    eye = jnp.eye(D, dtype=jnp.float32)
    ws_ext = jnp.concatenate([Ws[1:], eye[None]], axis=0)
    wn_ext = jnp.concatenate([Wn[1:], jnp.zeros((1, D, D), jnp.float32)], axis=0)
    b_ext = jnp.concatenate([b[1:], jnp.zeros((1, D), jnp.float32)], axis=0)

    p, q = _tc_first(x, Ws[0], Wn[0], b[0:1])

    def layer(i, pq):
        p, q = pq
        r = _segsum_sc(q.reshape(NQ * N, QH), srcq, dst_b)
        ws = lax.dynamic_index_in_dim(ws_ext, i, keepdims=False)
        wn = lax.dynamic_index_in_dim(wn_ext, i, keepdims=False)
        bi = lax.dynamic_index_in_dim(b_ext, i, keepdims=True)
        flag = jnp.where(i < 3, 1.0, 0.0).reshape(1, 1).astype(jnp.float32)
        return _tc_fuse(p, r, deg, ws, wn, bi, flag)

    p, q = lax.fori_loop(0, 4, layer, (p, q))
    return p


# speedup vs baseline: 1.0479x; 1.0479x over previous
import functools

import jax
import jax.numpy as jnp
from jax import lax
from jax.experimental import pallas as pl
from jax.experimental.pallas import tpu as pltpu
from jax.experimental.pallas import tpu_sc as plsc

N = 50000
D = 64
QH = 16
NQ = D // QH
NC, NS = 2, 16

K = 128
CPS = 16
SUP = K * CPS
NSUP = 26
E_T = SUP * NSUP
E_PAD = E_T * NS

AGG_ROWS = N + 8
NR = N // NC
HIST_ROWS = NR + 8

BR = 2000


def _zero_rows(ref, n_cols16):
    def body(r, _):
        for j in range(n_cols16):
            ref[r, pl.ds(16 * j, 16)] = jnp.zeros((16,), jnp.float32)
        return 0
    lax.fori_loop(0, K, body, 0)


@functools.partial(
    pl.kernel,
    out_type=jax.ShapeDtypeStruct((NQ * N, QH), jnp.float32),
    mesh=plsc.VectorSubcoreMesh(core_axis_name="c", subcore_axis_name="s"),
    scratch_types=[
        pltpu.VMEM((2, CPS, K), jnp.int32),
        pltpu.VMEM((2, CPS, K), jnp.int32),
        pltpu.VMEM((2, CPS, K, QH), jnp.float32),
        pltpu.VMEM((K, QH), jnp.float32),
        pltpu.VMEM_SHARED((AGG_ROWS, QH), jnp.float32),
        pltpu.SemaphoreType.DMA,
        pltpu.SemaphoreType.DMA,
        pltpu.SemaphoreType.DMA,
        pltpu.SemaphoreType.DMA,
    ],
    compiler_params=pltpu.CompilerParams(use_tc_tiling_on_sc=False),
)
def _segsum_sc(q, srcs, dstb, rout, sv, dv, rows, zrow, agg, g0, g1, s0, s1):
    c = lax.axis_index("c")
    s = lax.axis_index("s")
    gsem = (g0, g1)
    ssem = (s0, s1)

    _zero_rows(zrow, 1)
    base_z = jnp.minimum(s * 3200, AGG_ROWS - 25 * K)
    base_w = jnp.minimum(s * 3128, N - 3128)

    for qi in range(2):
        qt = c * 2 + qi
        qcols = pl.ds(qt * QH, QH)

        def fire_gather(sup, b):
            pltpu.sync_copy(srcs.at[qt, s, sup], sv.at[b])
            pltpu.sync_copy(dstb.at[s, sup], dv.at[b])
            for j in range(CPS):
                pltpu.async_copy(q.at[sv.at[b, j]], rows.at[b, j], gsem[b])

        def drain_gather(b):
            for j in range(CPS):
                pltpu.make_async_copy(q.at[sv.at[b, j]], rows.at[b, j],
                                      gsem[b]).wait()

        def fire_scatter(b):
            for j in range(CPS):
                pltpu.async_copy(rows.at[b, j], agg.at[dv.at[b, j]], ssem[b],
                                 add=True)

        def drain_scatter(b):
            for j in range(CPS):
                pltpu.make_async_copy(rows.at[b, j], agg.at[dv.at[b, j]],
                                      ssem[b]).wait()

        def zero_chunk(k, _):
            pltpu.sync_copy(zrow, agg.at[pl.ds(base_z + k * K, K)])
            return 0
        lax.fori_loop(0, 25, zero_chunk, 0)
        plsc.subcore_barrier()

        fire_gather(0, 0)
        fire_gather(1, 1)

        def pair(g2, _):
            for b in (0, 1):
                sup = 2 * g2 + b
                drain_gather(b)
                fire_scatter(b)
                drain_scatter(b)

                @pl.when(sup + 2 < NSUP)
                def _():
                    fire_gather(sup + 2, b)
            return 0
        lax.fori_loop(0, NSUP // 2, pair, 0)
        plsc.subcore_barrier()

        pltpu.sync_copy(agg.at[pl.ds(base_w, 3128)],
                        rout.at[pl.ds(qt * N + base_w, 3128)])
        plsc.subcore_barrier()


@functools.partial(
    pl.kernel,
    out_type=jax.ShapeDtypeStruct((N, 16), jnp.float32),
    mesh=plsc.VectorSubcoreMesh(core_axis_name="c", subcore_axis_name="s"),
    scratch_types=[
        pltpu.VMEM((2, CPS, K), jnp.int32),
        pltpu.VMEM((SUP, 16), jnp.float32),
        pltpu.VMEM((K, 16), jnp.float32),
        pltpu.VMEM_SHARED((HIST_ROWS, 16), jnp.float32),
        pltpu.SemaphoreType.DMA,
        pltpu.SemaphoreType.DMA,
    ],
    compiler_params=pltpu.CompilerParams(use_tc_tiling_on_sc=False),
)
def _deg_sc(dstb, dout, dv, onesb, zrow, hist, s0, s1):
    c = lax.axis_index("c")
    s = lax.axis_index("s")
    ssem = (s0, s1)

    def fill_ones(r, _):
        onesb[r, pl.ds(0, 16)] = jnp.ones((16,), jnp.float32)
        return 0
    lax.fori_loop(0, SUP, fill_ones, 0)
    _zero_rows(zrow, 1)

    base_z = jnp.minimum(s * 1568, HIST_ROWS - 13 * K)

    def zero_chunk(k, _):
        pltpu.sync_copy(zrow, hist.at[pl.ds(base_z + k * K, K)])
        return 0
    lax.fori_loop(0, 13, zero_chunk, 0)
    plsc.subcore_barrier()

    lo = c * NR

    def load_remap(sup, b):
        pltpu.sync_copy(dstb.at[s, sup], dv.at[b])

        def remap(t, _):
            def remap16(u, _):
                v = dv[b, t, pl.ds(u * 16, 16)] - lo
                ok = (v >= 0) & (v < NR)
                dv[b, t, pl.ds(u * 16, 16)] = jnp.where(ok, v, NR)
                return 0
            lax.fori_loop(0, K // 16, remap16, 0)
            return 0
        lax.fori_loop(0, CPS, remap, 0)

    def fire_scatter(b):
        for j in range(CPS):
            pltpu.async_copy(onesb.at[pl.ds(j * K, K)], hist.at[dv.at[b, j]],
                             ssem[b], add=True)

    def drain_scatter(b):
        for j in range(CPS):
            pltpu.make_async_copy(onesb.at[pl.ds(j * K, K)],
                                  hist.at[dv.at[b, j]], ssem[b]).wait()

    load_remap(0, 0)

    def pair(g2, _):
        for b in (0, 1):
            sup = 2 * g2 + b
            if b == 0:
                fire_scatter(0)
                load_remap(sup + 1, 1)
                drain_scatter(0)
            else:
                fire_scatter(1)

                @pl.when(sup + 1 < NSUP)
                def _():
                    load_remap(sup + 1, 0)
                drain_scatter(1)
        return 0
    lax.fori_loop(0, NSUP // 2, pair, 0)
    plsc.subcore_barrier()

    base_w = jnp.minimum(s * 1568, NR - 1568)
    pltpu.sync_copy(hist.at[pl.ds(base_w, 1568)],
                    dout.at[pl.ds(c * NR + base_w, 1568)])


def _store_quarters(q_ref, q):
    for k in range(NQ):
        q_ref[k] = q[:, QH * k:QH * (k + 1)]


def _mm_first_body(x_ref, ws_ref, wn_ref, b_ref, p_ref, q_ref):
    h = x_ref[...]
    p_ref[...] = (jnp.dot(h, ws_ref[...], preferred_element_type=jnp.float32)
                  + b_ref[...])
    _store_quarters(q_ref, jnp.dot(h, wn_ref[...],
                                   preferred_element_type=jnp.float32))


def _mean_from(r_ref, d_ref):
    deg = jnp.maximum(d_ref[:, 0:1], 1.0)
    m = jnp.concatenate([r_ref[k] for k in range(NQ)], axis=1)
    return m / deg


def _fuse_body(p_ref, r_ref, d_ref, ws_ref, wn_ref, b_ref, f_ref, po_ref, qo_ref):
    z = p_ref[...] + _mean_from(r_ref, d_ref)
    h = jnp.where(f_ref[0, 0] > 0.0, jnp.maximum(z, 0.0), z)
    po_ref[...] = (jnp.dot(h, ws_ref[...], preferred_element_type=jnp.float32)
                   + b_ref[...])
    _store_quarters(qo_ref, jnp.dot(h, wn_ref[...],
                                    preferred_element_type=jnp.float32))


_P_SPEC = pl.BlockSpec((BR, D), lambda i: (i, 0))
_Q_SPEC = pl.BlockSpec((NQ, BR, QH), lambda i: (0, i, 0))
_DEG_SPEC = pl.BlockSpec((BR, 16), lambda i: (i, 0))
_W_SPEC = pl.BlockSpec((D, D), lambda i: (0, 0))
_B_SPEC = pl.BlockSpec((1, D), lambda i: (0, 0))
_F_SPEC = pl.BlockSpec((1, 1), lambda i: (0, 0))
_PQ_SHAPES = [jax.ShapeDtypeStruct((N, D), jnp.float32),
              jax.ShapeDtypeStruct((NQ, N, QH), jnp.float32)]


def _tc_first(x, ws, wn, b1):
    return pl.pallas_call(
        _mm_first_body,
        grid=(N // BR,),
        in_specs=[_P_SPEC, _W_SPEC, _W_SPEC, _B_SPEC],
        out_specs=[_P_SPEC, _Q_SPEC],
        out_shape=_PQ_SHAPES,
    )(x, ws, wn, b1)


def _tc_fuse(p, r, deg, ws, wn, b1, flag):
    return pl.pallas_call(
        _fuse_body,
        grid=(N // BR,),
        in_specs=[_P_SPEC, _Q_SPEC, _DEG_SPEC, _W_SPEC, _W_SPEC, _B_SPEC,
                  _F_SPEC],
        out_specs=[_P_SPEC, _Q_SPEC],
        out_shape=_PQ_SHAPES,
    )(p, r, deg, ws, wn, b1, flag)


def kernel(x, edge_index, Ws, Wn, b):
    src = edge_index[0].astype(jnp.int32)
    dst = edge_index[1].astype(jnp.int32)
    pad = E_PAD - src.shape[0]
    src_p = jnp.concatenate([src, jnp.zeros((pad,), jnp.int32)])
    dst_p = jnp.concatenate([dst, jnp.full((pad,), N, jnp.int32)])
    srcq = (src_p[None, :]
            + (jnp.arange(NQ, dtype=jnp.int32) * N)[:, None]
            ).reshape(NQ, NS, NSUP, CPS, K)
    dst_b = dst_p.reshape(NS, NSUP, CPS, K)

    deg = _deg_sc(dst_b)
    eye = jnp.eye(D, dtype=jnp.float32)
    ws_ext = jnp.concatenate([Ws[1:], eye[None]], axis=0)
    wn_ext = jnp.concatenate([Wn[1:], jnp.zeros((1, D, D), jnp.float32)], axis=0)
    b_ext = jnp.concatenate([b[1:], jnp.zeros((1, D), jnp.float32)], axis=0)

    p, q = _tc_first(x, Ws[0], Wn[0], b[0:1])

    def layer(i, pq):
        p, q = pq
        r = _segsum_sc(q.reshape(NQ * N, QH), srcq, dst_b)
        r = r.reshape(NQ, N, QH)
        ws = lax.dynamic_index_in_dim(ws_ext, i, keepdims=False)
        wn = lax.dynamic_index_in_dim(wn_ext, i, keepdims=False)
        bi = lax.dynamic_index_in_dim(b_ext, i, keepdims=True)
        flag = jnp.where(i < 3, 1.0, 0.0).reshape(1, 1).astype(jnp.float32)
        return _tc_fuse(p, r, deg, ws, wn, bi, flag)

    p, q = lax.fori_loop(0, 4, layer, (p, q))
    return p


# speedup vs baseline: 1.3418x; 1.2805x over previous
import functools

import jax
import jax.numpy as jnp
from jax import lax
from jax.experimental import pallas as pl
from jax.experimental.pallas import tpu as pltpu
from jax.experimental.pallas import tpu_sc as plsc

N = 50000
D = 64
QH = 16
NQ = D // QH
NC, NS = 2, 16

K = 128
CPS = 16
SUP = K * CPS
NSUP = 25
E_T = SUP * NSUP
E_PAD = E_T * NS

AGG_ROWS = N + 8
NR = N // NC
HIST_ROWS = NR + 8

BR = 2000


def _zero_rows(ref, n_cols16):
    def body(r, _):
        for j in range(n_cols16):
            ref[r, pl.ds(16 * j, 16)] = jnp.zeros((16,), jnp.float32)
        return 0
    lax.fori_loop(0, K, body, 0)


@functools.partial(
    pl.kernel,
    out_type=jax.ShapeDtypeStruct((NQ * N, QH), jnp.float32),
    mesh=plsc.VectorSubcoreMesh(core_axis_name="c", subcore_axis_name="s"),
    scratch_types=[
        pltpu.VMEM((CPS, K), jnp.int32),
        pltpu.VMEM((CPS, K), jnp.int32),
        pltpu.VMEM((CPS, K, QH), jnp.float32),
        pltpu.VMEM((K, QH), jnp.float32),
        pltpu.VMEM_SHARED((AGG_ROWS, QH), jnp.float32),
        pltpu.SemaphoreType.DMA,
    ],
    compiler_params=pltpu.CompilerParams(use_tc_tiling_on_sc=False),
)
def _segsum_sc(q4, srcs, dstb, rout, sv, dv, rows, zrow, agg, sem):
    c = lax.axis_index("c")
    s = lax.axis_index("s")

    _zero_rows(zrow, 1)
    base_z = jnp.minimum(s * 3200, AGG_ROWS - 25 * K)
    base_w = jnp.minimum(s * 3128, N - 3128)

    for qi in range(2):
        qt = c * 2 + qi
        base_g = qt * N

        def zero_chunk(k, _):
            pltpu.sync_copy(zrow, agg.at[pl.ds(base_z + k * K, K)])
            return 0
        lax.fori_loop(0, 25, zero_chunk, 0)
        plsc.subcore_barrier()

        def super_chunk(sup, _):
            pltpu.sync_copy(srcs.at[s, sup], sv)
            pltpu.sync_copy(dstb.at[s, sup], dv)

            def add_base(t, _):
                def add16(u, _):
                    sv[t, pl.ds(u * 16, 16)] = sv[t, pl.ds(u * 16, 16)] + base_g
                    return 0
                lax.fori_loop(0, K // 16, add16, 0)
                return 0
            lax.fori_loop(0, CPS, add_base, 0)

            copies = []
            for j in range(CPS):
                copies.append(pltpu.async_copy(q4.at[sv.at[j]], rows.at[j], sem))
            for cp in copies:
                cp.wait()
            for j in range(CPS):
                pltpu.sync_copy(rows.at[j], agg.at[dv.at[j]], add=True)
            return 0
        lax.fori_loop(0, NSUP, super_chunk, 0)
        plsc.subcore_barrier()

        pltpu.sync_copy(agg.at[pl.ds(base_w, 3128)],
                        rout.at[pl.ds(base_g + base_w, 3128)])
        plsc.subcore_barrier()


@functools.partial(
    pl.kernel,
    out_type=jax.ShapeDtypeStruct((N, 16), jnp.float32),
    mesh=plsc.VectorSubcoreMesh(core_axis_name="c", subcore_axis_name="s"),
    scratch_types=[
        pltpu.VMEM((CPS, K), jnp.int32),
        pltpu.VMEM((K, 16), jnp.float32),
        pltpu.VMEM((K, 16), jnp.float32),
        pltpu.VMEM_SHARED((HIST_ROWS, 16), jnp.float32),
    ],
    compiler_params=pltpu.CompilerParams(use_tc_tiling_on_sc=False),
)
def _deg_sc(dstb, dout, dv, ones_v, zrow, hist):
    c = lax.axis_index("c")
    s = lax.axis_index("s")

    def fill_ones(r, _):
        ones_v[r, pl.ds(0, 16)] = jnp.ones((16,), jnp.float32)
        return 0
    lax.fori_loop(0, K, fill_ones, 0)
    _zero_rows(zrow, 1)

    base_z = jnp.minimum(s * 1568, HIST_ROWS - 13 * K)

    def zero_chunk(k, _):
        pltpu.sync_copy(zrow, hist.at[pl.ds(base_z + k * K, K)])
        return 0
    lax.fori_loop(0, 13, zero_chunk, 0)
    plsc.subcore_barrier()

    lo = c * NR

    def super_chunk(sup, _):
        pltpu.sync_copy(dstb.at[s, sup], dv)

        def remap(t, _):
            def remap16(u, _):
                v = dv[t, pl.ds(u * 16, 16)] - lo
                ok = (v >= 0) & (v < NR)
                dv[t, pl.ds(u * 16, 16)] = jnp.where(ok, v, NR)
                return 0
            lax.fori_loop(0, K // 16, remap16, 0)
            return 0
        lax.fori_loop(0, CPS, remap, 0)
        for j in range(CPS):
            pltpu.sync_copy(ones_v, hist.at[dv.at[j]], add=True)
        return 0
    lax.fori_loop(0, NSUP, super_chunk, 0)
    plsc.subcore_barrier()

    base_w = jnp.minimum(s * 1568, NR - 1568)
    pltpu.sync_copy(hist.at[pl.ds(base_w, 1568)],
                    dout.at[pl.ds(c * NR + base_w, 1568)])


def _store_quarters(q_ref, q):
    for k in range(NQ):
        q_ref[k] = q[:, QH * k:QH * (k + 1)]


def _mm_first_body(x_ref, ws_ref, wn_ref, b_ref, p_ref, q_ref):
    h = x_ref[...]
    p_ref[...] = (jnp.dot(h, ws_ref[...], preferred_element_type=jnp.float32)
                  + b_ref[...])
    _store_quarters(q_ref, jnp.dot(h, wn_ref[...],
                                   preferred_element_type=jnp.float32))


def _mean_from(r_ref, d_ref):
    deg = jnp.maximum(d_ref[:, 0:1], 1.0)
    m = jnp.concatenate([r_ref[k] for k in range(NQ)], axis=1)
    return m / deg


def _fuse_body(p_ref, r_ref, d_ref, ws_ref, wn_ref, b_ref, f_ref, po_ref, qo_ref):
    z = p_ref[...] + _mean_from(r_ref, d_ref)
    h = jnp.where(f_ref[0, 0] > 0.0, jnp.maximum(z, 0.0), z)
    po_ref[...] = (jnp.dot(h, ws_ref[...], preferred_element_type=jnp.float32)
                   + b_ref[...])
    _store_quarters(qo_ref, jnp.dot(h, wn_ref[...],
                                    preferred_element_type=jnp.float32))


_P_SPEC = pl.BlockSpec((BR, D), lambda i: (i, 0))
_Q_SPEC = pl.BlockSpec((NQ, BR, QH), lambda i: (0, i, 0))
_DEG_SPEC = pl.BlockSpec((BR, 16), lambda i: (i, 0))
_W_SPEC = pl.BlockSpec((D, D), lambda i: (0, 0))
_B_SPEC = pl.BlockSpec((1, D), lambda i: (0, 0))
_F_SPEC = pl.BlockSpec((1, 1), lambda i: (0, 0))
_PQ_SHAPES = [jax.ShapeDtypeStruct((N, D), jnp.float32),
              jax.ShapeDtypeStruct((NQ, N, QH), jnp.float32)]


def _tc_first(x, ws, wn, b1):
    return pl.pallas_call(
        _mm_first_body,
        grid=(N // BR,),
        in_specs=[_P_SPEC, _W_SPEC, _W_SPEC, _B_SPEC],
        out_specs=[_P_SPEC, _Q_SPEC],
        out_shape=_PQ_SHAPES,
    )(x, ws, wn, b1)


def _tc_fuse(p, r, deg, ws, wn, b1, flag):
    return pl.pallas_call(
        _fuse_body,
        grid=(N // BR,),
        in_specs=[_P_SPEC, _Q_SPEC, _DEG_SPEC, _W_SPEC, _W_SPEC, _B_SPEC,
                  _F_SPEC],
        out_specs=[_P_SPEC, _Q_SPEC],
        out_shape=_PQ_SHAPES,
    )(p, r, deg, ws, wn, b1, flag)


def kernel(x, edge_index, Ws, Wn, b):
    src = edge_index[0].astype(jnp.int32)
    dst = edge_index[1].astype(jnp.int32)
    pad = E_PAD - src.shape[0]
    src_p = jnp.concatenate([src, jnp.zeros((pad,), jnp.int32)])
    dst_p = jnp.concatenate([dst, jnp.full((pad,), N, jnp.int32)])
    srcs = src_p.reshape(NS, NSUP, CPS, K)
    dst_b = dst_p.reshape(NS, NSUP, CPS, K)

    deg = _deg_sc(dst_b)
    eye = jnp.eye(D, dtype=jnp.float32)
    ws_ext = jnp.concatenate([Ws[1:], eye[None]], axis=0)
    wn_ext = jnp.concatenate([Wn[1:], jnp.zeros((1, D, D), jnp.float32)], axis=0)
    b_ext = jnp.concatenate([b[1:], jnp.zeros((1, D), jnp.float32)], axis=0)

    p, q = _tc_first(x, Ws[0], Wn[0], b[0:1])

    def layer(i, pq):
        p, q = pq
        r = _segsum_sc(q.reshape(NQ * N, QH), srcs, dst_b).reshape(NQ, N, QH)
        ws = lax.dynamic_index_in_dim(ws_ext, i, keepdims=False)
        wn = lax.dynamic_index_in_dim(wn_ext, i, keepdims=False)
        bi = lax.dynamic_index_in_dim(b_ext, i, keepdims=True)
        flag = jnp.where(i < 3, 1.0, 0.0).reshape(1, 1).astype(jnp.float32)
        return _tc_fuse(p, r, deg, ws, wn, bi, flag)

    p, q = lax.fori_loop(0, 4, layer, (p, q))
    return p


# speedup vs baseline: 2.1320x; 1.5889x over previous
import functools

import jax
import jax.numpy as jnp
from jax import lax
from jax.experimental import pallas as pl
from jax.experimental.pallas import tpu as pltpu
from jax.experimental.pallas import tpu_sc as plsc

N = 50000
D = 64
HB = 32
NH = D // HB
NC, NS = 2, 16

K = 128
CPS = 16
SUP = K * CPS
NSUP = 25
E_T = SUP * NSUP
E_PAD = E_T * NS

AGG_ROWS = N + 8
NR = N // NC
HIST_ROWS = NR + 8

BR = 2000


def _zero_rows(ref, n_cols16):
    def body(r, _):
        for j in range(n_cols16):
            ref[r, pl.ds(16 * j, 16)] = jnp.zeros((16,), jnp.float32)
        return 0
    lax.fori_loop(0, K, body, 0)


@functools.partial(
    pl.kernel,
    out_type=jax.ShapeDtypeStruct((NC * N, HB), jnp.bfloat16),
    mesh=plsc.VectorSubcoreMesh(core_axis_name="c", subcore_axis_name="s"),
    scratch_types=[
        pltpu.VMEM((CPS, K), jnp.int32),
        pltpu.VMEM((CPS, K), jnp.int32),
        pltpu.VMEM((CPS, K, HB), jnp.bfloat16),
        pltpu.VMEM((K, HB), jnp.bfloat16),
        pltpu.VMEM_SHARED((AGG_ROWS, HB), jnp.bfloat16),
        pltpu.SemaphoreType.DMA,
    ],
    compiler_params=pltpu.CompilerParams(use_tc_tiling_on_sc=False),
)
def _segsum_sc(q2, srcs, dstb, rout, sv, dv, rows, zrow, agg, sem):
    c = lax.axis_index("c")
    s = lax.axis_index("s")

    def zero_z(r, _):
        zrow[r, pl.ds(0, 32)] = jnp.zeros((32,), jnp.bfloat16)
        return 0
    lax.fori_loop(0, K, zero_z, 0)
    base_z = jnp.minimum(s * 3200, AGG_ROWS - 25 * K)
    base_w = jnp.minimum(s * 3128, N - 3128)
    base_g = c * N

    def zero_chunk(k, _):
        pltpu.sync_copy(zrow, agg.at[pl.ds(base_z + k * K, K)])
        return 0
    lax.fori_loop(0, 25, zero_chunk, 0)
    plsc.subcore_barrier()

    def super_chunk(sup, _):
        pltpu.sync_copy(srcs.at[s, sup], sv)
        pltpu.sync_copy(dstb.at[s, sup], dv)

        def add_base(t, _):
            def add16(u, _):
                sv[t, pl.ds(u * 16, 16)] = sv[t, pl.ds(u * 16, 16)] + base_g
                return 0
            lax.fori_loop(0, K // 16, add16, 0)
            return 0
        lax.fori_loop(0, CPS, add_base, 0)

        copies = []
        for j in range(CPS):
            copies.append(pltpu.async_copy(q2.at[sv.at[j]], rows.at[j], sem))
        for cp in copies:
            cp.wait()
        for j in range(CPS):
            pltpu.sync_copy(rows.at[j], agg.at[dv.at[j]], add=True)
        return 0
    lax.fori_loop(0, NSUP, super_chunk, 0)
    plsc.subcore_barrier()

    pltpu.sync_copy(agg.at[pl.ds(base_w, 3128)],
                    rout.at[pl.ds(c * N + base_w, 3128)])


@functools.partial(
    pl.kernel,
    out_type=jax.ShapeDtypeStruct((N, 16), jnp.float32),
    mesh=plsc.VectorSubcoreMesh(core_axis_name="c", subcore_axis_name="s"),
    scratch_types=[
        pltpu.VMEM((CPS, K), jnp.int32),
        pltpu.VMEM((K, 16), jnp.float32),
        pltpu.VMEM((K, 16), jnp.float32),
        pltpu.VMEM_SHARED((HIST_ROWS, 16), jnp.float32),
    ],
    compiler_params=pltpu.CompilerParams(use_tc_tiling_on_sc=False),
)
def _deg_sc(dstb, dout, dv, ones_v, zrow, hist):
    c = lax.axis_index("c")
    s = lax.axis_index("s")

    def fill_ones(r, _):
        ones_v[r, pl.ds(0, 16)] = jnp.ones((16,), jnp.float32)
        return 0
    lax.fori_loop(0, K, fill_ones, 0)
    _zero_rows(zrow, 1)

    base_z = jnp.minimum(s * 1568, HIST_ROWS - 13 * K)

    def zero_chunk(k, _):
        pltpu.sync_copy(zrow, hist.at[pl.ds(base_z + k * K, K)])
        return 0
    lax.fori_loop(0, 13, zero_chunk, 0)
    plsc.subcore_barrier()

    lo = c * NR

    def super_chunk(sup, _):
        pltpu.sync_copy(dstb.at[s, sup], dv)

        def remap(t, _):
            def remap16(u, _):
                v = dv[t, pl.ds(u * 16, 16)] - lo
                ok = (v >= 0) & (v < NR)
                dv[t, pl.ds(u * 16, 16)] = jnp.where(ok, v, NR)
                return 0
            lax.fori_loop(0, K // 16, remap16, 0)
            return 0
        lax.fori_loop(0, CPS, remap, 0)
        for j in range(CPS):
            pltpu.sync_copy(ones_v, hist.at[dv.at[j]], add=True)
        return 0
    lax.fori_loop(0, NSUP, super_chunk, 0)
    plsc.subcore_barrier()

    base_w = jnp.minimum(s * 1568, NR - 1568)
    pltpu.sync_copy(hist.at[pl.ds(base_w, 1568)],
                    dout.at[pl.ds(c * NR + base_w, 1568)])


def _store_halves(q_ref, q):
    qb = q.astype(jnp.bfloat16)
    q_ref[0] = qb[:, :HB]
    q_ref[1] = qb[:, HB:]


def _mm_first_body(x_ref, ws_ref, wn_ref, b_ref, p_ref, q_ref):
    h = x_ref[...]
    p_ref[...] = (jnp.dot(h, ws_ref[...], preferred_element_type=jnp.float32)
                  + b_ref[...])
    _store_halves(q_ref, jnp.dot(h, wn_ref[...],
                                  preferred_element_type=jnp.float32))


def _mean_from(r_ref, d_ref):
    deg = jnp.maximum(d_ref[:, 0:1], 1.0)
    m = jnp.concatenate([r_ref[0], r_ref[1]], axis=1).astype(jnp.float32)
    return m / deg


def _fuse_body(p_ref, r_ref, d_ref, ws_ref, wn_ref, b_ref, f_ref, po_ref, qo_ref):
    z = p_ref[...] + _mean_from(r_ref, d_ref)
    h = jnp.where(f_ref[0, 0] > 0.0, jnp.maximum(z, 0.0), z)
    po_ref[...] = (jnp.dot(h, ws_ref[...], preferred_element_type=jnp.float32)
                   + b_ref[...])
    _store_halves(qo_ref, jnp.dot(h, wn_ref[...],
                                   preferred_element_type=jnp.float32))


_P_SPEC = pl.BlockSpec((BR, D), lambda i: (i, 0))
_Q_SPEC = pl.BlockSpec((NH, BR, HB), lambda i: (0, i, 0))
_DEG_SPEC = pl.BlockSpec((BR, 16), lambda i: (i, 0))
_W_SPEC = pl.BlockSpec((D, D), lambda i: (0, 0))
_B_SPEC = pl.BlockSpec((1, D), lambda i: (0, 0))
_F_SPEC = pl.BlockSpec((1, 1), lambda i: (0, 0))
_PQ_SHAPES = [jax.ShapeDtypeStruct((N, D), jnp.float32),
              jax.ShapeDtypeStruct((NH, N, HB), jnp.bfloat16)]


def _tc_first(x, ws, wn, b1):
    return pl.pallas_call(
        _mm_first_body,
        grid=(N // BR,),
        in_specs=[_P_SPEC, _W_SPEC, _W_SPEC, _B_SPEC],
        out_specs=[_P_SPEC, _Q_SPEC],
        out_shape=_PQ_SHAPES,
    )(x, ws, wn, b1)


def _tc_fuse(p, r, deg, ws, wn, b1, flag):
    return pl.pallas_call(
        _fuse_body,
        grid=(N // BR,),
        in_specs=[_P_SPEC, _Q_SPEC, _DEG_SPEC, _W_SPEC, _W_SPEC, _B_SPEC,
                  _F_SPEC],
        out_specs=[_P_SPEC, _Q_SPEC],
        out_shape=_PQ_SHAPES,
    )(p, r, deg, ws, wn, b1, flag)


def kernel(x, edge_index, Ws, Wn, b):
    src = edge_index[0].astype(jnp.int32)
    dst = edge_index[1].astype(jnp.int32)
    pad = E_PAD - src.shape[0]
    src_p = jnp.concatenate([src, jnp.zeros((pad,), jnp.int32)])
    dst_p = jnp.concatenate([dst, jnp.full((pad,), N, jnp.int32)])
    srcs = src_p.reshape(NS, NSUP, CPS, K)
    dst_b = dst_p.reshape(NS, NSUP, CPS, K)

    deg = _deg_sc(dst_b)
    eye = jnp.eye(D, dtype=jnp.float32)
    ws_ext = jnp.concatenate([Ws[1:], eye[None]], axis=0)
    wn_ext = jnp.concatenate([Wn[1:], jnp.zeros((1, D, D), jnp.float32)], axis=0)
    b_ext = jnp.concatenate([b[1:], jnp.zeros((1, D), jnp.float32)], axis=0)

    p, q = _tc_first(x, Ws[0], Wn[0], b[0:1])

    def layer(i, pq):
        p, q = pq
        r = _segsum_sc(q.reshape(NH * N, HB), srcs, dst_b).reshape(NH, N, HB)
        ws = lax.dynamic_index_in_dim(ws_ext, i, keepdims=False)
        wn = lax.dynamic_index_in_dim(wn_ext, i, keepdims=False)
        bi = lax.dynamic_index_in_dim(b_ext, i, keepdims=True)
        flag = jnp.where(i < 3, 1.0, 0.0).reshape(1, 1).astype(jnp.float32)
        return _tc_fuse(p, r, deg, ws, wn, bi, flag)

    p, q = lax.fori_loop(0, 4, layer, (p, q))
    return p


# speedup vs baseline: 2.1360x; 1.0019x over previous
import functools

import jax
import jax.numpy as jnp
from jax import lax
from jax.experimental import pallas as pl
from jax.experimental.pallas import tpu as pltpu
from jax.experimental.pallas import tpu_sc as plsc

N = 50000
D = 64
HB = 32
NH = D // HB
NC, NS = 2, 16

K = 128
CPS = 16
SUP = K * CPS
NSUP = 25
E_T = SUP * NSUP
E_PAD = E_T * NS

AGG_ROWS = N + 8
NR = N // NC
HIST_ROWS = NR + 8

BR = 2000


def _zero_rows(ref, n_cols16):
    def body(r, _):
        for j in range(n_cols16):
            ref[r, pl.ds(16 * j, 16)] = jnp.zeros((16,), jnp.float32)
        return 0
    lax.fori_loop(0, K, body, 0)


@functools.partial(
    pl.kernel,
    out_type=jax.ShapeDtypeStruct((NC * N, HB), jnp.bfloat16),
    mesh=plsc.VectorSubcoreMesh(core_axis_name="c", subcore_axis_name="s"),
    scratch_types=[
        pltpu.VMEM((CPS, K), jnp.int32),
        pltpu.VMEM((CPS, K), jnp.int32),
        pltpu.VMEM((CPS, K, HB), jnp.bfloat16),
        pltpu.VMEM((K, HB), jnp.bfloat16),
        pltpu.VMEM_SHARED((AGG_ROWS, HB), jnp.bfloat16),
        pltpu.SemaphoreType.DMA,
    ],
    compiler_params=pltpu.CompilerParams(use_tc_tiling_on_sc=False),
)
def _segsum_sc(q2, srcs, dstb, rout, sv, dv, rows, zrow, agg, sem):
    c = lax.axis_index("c")
    s = lax.axis_index("s")

    def zero_z(r, _):
        zrow[r, pl.ds(0, 32)] = jnp.zeros((32,), jnp.bfloat16)
        return 0
    lax.fori_loop(0, K, zero_z, 0)
    base_z = jnp.minimum(s * 3200, AGG_ROWS - 25 * K)
    base_w = jnp.minimum(s * 3128, N - 3128)
    base_g = c * N

    def zero_chunk(k, _):
        pltpu.sync_copy(zrow, agg.at[pl.ds(base_z + k * K, K)])
        return 0
    lax.fori_loop(0, 25, zero_chunk, 0)
    plsc.subcore_barrier()

    def super_chunk(sup, _):
        pltpu.sync_copy(srcs.at[s, sup], sv)
        pltpu.sync_copy(dstb.at[s, sup], dv)

        def add_base(t, _):
            def add16(u, _):
                sv[t, pl.ds(u * 16, 16)] = sv[t, pl.ds(u * 16, 16)] + base_g
                return 0
            lax.fori_loop(0, K // 16, add16, 0)
            return 0
        lax.fori_loop(0, CPS, add_base, 0)

        copies = []
        for j in range(CPS):
            copies.append(pltpu.async_copy(q2.at[sv.at[j]], rows.at[j], sem))
        for cp in copies:
            cp.wait()
        for j in range(CPS):
            pltpu.sync_copy(rows.at[j], agg.at[dv.at[j]], add=True)
        return 0
    lax.fori_loop(0, NSUP, super_chunk, 0)
    plsc.subcore_barrier()

    pltpu.sync_copy(agg.at[pl.ds(base_w, 3128)],
                    rout.at[pl.ds(c * N + base_w, 3128)])


@functools.partial(
    pl.kernel,
    out_type=jax.ShapeDtypeStruct((N, 16), jnp.float32),
    mesh=plsc.VectorSubcoreMesh(core_axis_name="c", subcore_axis_name="s"),
    scratch_types=[
        pltpu.VMEM((CPS, K), jnp.int32),
        pltpu.VMEM((K, 16), jnp.float32),
        pltpu.VMEM((K, 16), jnp.float32),
        pltpu.VMEM_SHARED((HIST_ROWS, 16), jnp.float32),
    ],
    compiler_params=pltpu.CompilerParams(use_tc_tiling_on_sc=False),
)
def _deg_sc(dstd, dout, dv, ones_v, zrow, hist):
    c = lax.axis_index("c")
    s = lax.axis_index("s")

    def fill_ones(r, _):
        ones_v[r, pl.ds(0, 16)] = jnp.ones((16,), jnp.float32)
        return 0
    lax.fori_loop(0, K, fill_ones, 0)
    _zero_rows(zrow, 1)

    base_z = jnp.minimum(s * 1568, HIST_ROWS - 13 * K)

    def zero_chunk(k, _):
        pltpu.sync_copy(zrow, hist.at[pl.ds(base_z + k * K, K)])
        return 0
    lax.fori_loop(0, 13, zero_chunk, 0)
    plsc.subcore_barrier()

    def super_chunk(sup, _):
        pltpu.sync_copy(dstd.at[c, s, sup], dv)
        for j in range(CPS):
            pltpu.sync_copy(ones_v, hist.at[dv.at[j]], add=True)
        return 0
    lax.fori_loop(0, NSUP, super_chunk, 0)
    plsc.subcore_barrier()

    base_w = jnp.minimum(s * 1568, NR - 1568)
    pltpu.sync_copy(hist.at[pl.ds(base_w, 1568)],
                    dout.at[pl.ds(c * NR + base_w, 1568)])


def _store_halves(q_ref, q):
    qb = q.astype(jnp.bfloat16)
    q_ref[0] = qb[:, :HB]
    q_ref[1] = qb[:, HB:]


def _mm_first_body(x_ref, ws_ref, wn_ref, b_ref, p_ref, q_ref):
    h = x_ref[...]
    p_ref[...] = (jnp.dot(h, ws_ref[...], preferred_element_type=jnp.float32)
                  + b_ref[...])
    _store_halves(q_ref, jnp.dot(h, wn_ref[...],
                                  preferred_element_type=jnp.float32))


def _mean_from(r_ref, d_ref):
    deg = jnp.maximum(d_ref[:, 0:1], 1.0)
    m = jnp.concatenate([r_ref[0], r_ref[1]], axis=1).astype(jnp.float32)
    return m / deg


def _fuse_body(p_ref, r_ref, d_ref, ws_ref, wn_ref, b_ref, f_ref, po_ref, qo_ref):
    z = p_ref[...] + _mean_from(r_ref, d_ref)
    h = jnp.where(f_ref[0, 0] > 0.0, jnp.maximum(z, 0.0), z)
    po_ref[...] = (jnp.dot(h, ws_ref[...], preferred_element_type=jnp.float32)
                   + b_ref[...])
    _store_halves(qo_ref, jnp.dot(h, wn_ref[...],
                                   preferred_element_type=jnp.float32))


_P_SPEC = pl.BlockSpec((BR, D), lambda i: (i, 0))
_Q_SPEC = pl.BlockSpec((NH, BR, HB), lambda i: (0, i, 0))
_DEG_SPEC = pl.BlockSpec((BR, 16), lambda i: (i, 0))
_W_SPEC = pl.BlockSpec((D, D), lambda i: (0, 0))
_B_SPEC = pl.BlockSpec((1, D), lambda i: (0, 0))
_F_SPEC = pl.BlockSpec((1, 1), lambda i: (0, 0))
_PQ_SHAPES = [jax.ShapeDtypeStruct((N, D), jnp.float32),
              jax.ShapeDtypeStruct((NH, N, HB), jnp.bfloat16)]


def _tc_first(x, ws, wn, b1):
    return pl.pallas_call(
        _mm_first_body,
        grid=(N // BR,),
        in_specs=[_P_SPEC, _W_SPEC, _W_SPEC, _B_SPEC],
        out_specs=[_P_SPEC, _Q_SPEC],
        out_shape=_PQ_SHAPES,
    )(x, ws, wn, b1)


def _tc_fuse(p, r, deg, ws, wn, b1, flag):
    return pl.pallas_call(
        _fuse_body,
        grid=(N // BR,),
        in_specs=[_P_SPEC, _Q_SPEC, _DEG_SPEC, _W_SPEC, _W_SPEC, _B_SPEC,
                  _F_SPEC],
        out_specs=[_P_SPEC, _Q_SPEC],
        out_shape=_PQ_SHAPES,
    )(p, r, deg, ws, wn, b1, flag)


def kernel(x, edge_index, Ws, Wn, b):
    src = edge_index[0].astype(jnp.int32)
    dst = edge_index[1].astype(jnp.int32)
    pad = E_PAD - src.shape[0]
    src_p = jnp.concatenate([src, jnp.zeros((pad,), jnp.int32)])
    dst_p = jnp.concatenate([dst, jnp.full((pad,), N, jnp.int32)])
    srcs = src_p.reshape(NS, NSUP, CPS, K)
    dst_b = dst_p.reshape(NS, NSUP, CPS, K)
    lo = (jnp.arange(NC, dtype=jnp.int32) * NR)[:, None]
    dloc = dst_p[None, :] - lo
    dst_d = jnp.where((dloc >= 0) & (dloc < NR), dloc, NR
                      ).reshape(NC, NS, NSUP, CPS, K)

    deg = _deg_sc(dst_d)
    eye = jnp.eye(D, dtype=jnp.float32)
    ws_ext = jnp.concatenate([Ws[1:], eye[None]], axis=0)
    wn_ext = jnp.concatenate([Wn[1:], jnp.zeros((1, D, D), jnp.float32)], axis=0)
    b_ext = jnp.concatenate([b[1:], jnp.zeros((1, D), jnp.float32)], axis=0)

    p, q = _tc_first(x, Ws[0], Wn[0], b[0:1])

    def layer(i, pq):
        p, q = pq
        r = _segsum_sc(q.reshape(NH * N, HB), srcs, dst_b).reshape(NH, N, HB)
        ws = lax.dynamic_index_in_dim(ws_ext, i, keepdims=False)
        wn = lax.dynamic_index_in_dim(wn_ext, i, keepdims=False)
        bi = lax.dynamic_index_in_dim(b_ext, i, keepdims=True)
        flag = jnp.where(i < 3, 1.0, 0.0).reshape(1, 1).astype(jnp.float32)
        return _tc_fuse(p, r, deg, ws, wn, bi, flag)

    p, q = lax.fori_loop(0, 4, layer, (p, q))
    return p
